# Initial kernel scaffold; baseline (speedup 1.0000x reference)
import jax
import jax.numpy as jnp
from jax.experimental import pallas as pl


def kernel(x, W1_l, b1, W1_r, W2_l, b2, W2_r, W3_l, b3, W3_r, edge_index):
    raise NotImplementedError("write your pallas kernel here")


# speedup vs baseline: 4.1823x; 4.1823x over previous
import jax
import jax.numpy as jnp
from jax import lax
from jax.experimental import pallas as pl
from jax.experimental.pallas import tpu as pltpu
from jax.experimental.pallas import tpu_sc as plsc

NC = 2
NS = 16
L = 16
K = 128
IB = 8
WO = 160


def _make_segsum(n_nodes, n_pad, n_chunks, dh, with_cnt):
  rpt = n_pad // NS
  n_rounds = rpt // WO
  n_blocks = n_chunks // IB

  mesh = plsc.VectorSubcoreMesh(
      core_axis_name="c", subcore_axis_name="s",
      num_cores=NC, num_subcores=NS)

  def body(tbl, srcs, dsts, *rest):
    if with_cnt:
      (zrow, zc, oc, agg_h, cnt_h, src_v, dst_v, rows_v, out_v,
       ones_v, cnt_ov, agg_sh, cnt_sh, sem) = rest
    else:
      (zrow, agg_h, src_v, dst_v, rows_v, out_v, agg_sh, sem) = rest
    cid = lax.axis_index("c")
    sid = lax.axis_index("s")
    base = sid * rpt

    pltpu.sync_copy(zrow, out_v)
    for r in range(n_rounds):
      pltpu.sync_copy(out_v, agg_sh.at[pl.ds(base + r * WO, WO)])
    if with_cnt:
      pltpu.sync_copy(zc, cnt_ov)
      for r in range(n_rounds):
        pltpu.sync_copy(cnt_ov, cnt_sh.at[pl.ds(base + r * WO, WO)])
      pltpu.sync_copy(oc, ones_v)
    plsc.subcore_barrier()

    def block(b, carry):
      pltpu.sync_copy(srcs.at[cid, sid, pl.ds(b * IB, IB)], src_v)
      pltpu.sync_copy(dsts.at[sid, pl.ds(b * IB, IB)], dst_v)
      for i in range(IB):
        pltpu.async_copy(tbl.at[src_v.at[i]], rows_v, sem).wait()
        pltpu.sync_copy(rows_v, agg_sh.at[dst_v.at[i]], add=True)
        if with_cnt:
          pltpu.sync_copy(ones_v, cnt_sh.at[dst_v.at[i]], add=True)
      return carry
    lax.fori_loop(0, n_blocks, block, 0)
    plsc.subcore_barrier()

    for r in range(n_rounds):
      pltpu.sync_copy(agg_sh.at[pl.ds(base + r * WO, WO)], out_v)
      pltpu.sync_copy(out_v, agg_h.at[cid, pl.ds(base + r * WO, WO)])
    if with_cnt:
      @pl.when(cid == 0)
      def _():
        for r in range(n_rounds):
          pltpu.sync_copy(cnt_sh.at[pl.ds(base + r * WO, WO)], cnt_ov)
          pltpu.sync_copy(cnt_ov, cnt_h.at[pl.ds(base + r * WO, WO)])

  out_type = [jax.ShapeDtypeStruct((NC, n_pad, dh), jnp.float32)]
  scratch = [
      pltpu.VMEM((IB, K), jnp.int32),
      pltpu.VMEM((IB, K), jnp.int32),
      pltpu.VMEM((K, dh), jnp.float32),
      pltpu.VMEM((WO, dh), jnp.float32),
  ]
  if with_cnt:
    out_type.append(jax.ShapeDtypeStruct((n_pad, L), jnp.float32))
    scratch += [
        pltpu.VMEM((K, L), jnp.float32),
        pltpu.VMEM((WO, L), jnp.float32),
    ]
  scratch += [pltpu.VMEM_SHARED((n_pad, dh), jnp.float32)]
  if with_cnt:
    scratch += [pltpu.VMEM_SHARED((n_pad, L), jnp.float32)]
  scratch += [pltpu.SemaphoreType.DMA]

  return pl.kernel(
      body, out_type=tuple(out_type), mesh=mesh,
      scratch_types=tuple(scratch),
      compiler_params=pltpu.CompilerParams(use_tc_tiling_on_sc=False))


def _tc_layer(h, agg, cnt, wl, wr, b, bn, relu):
  _, n, dh = h.shape
  d_out = wl.shape[1]
  dho = d_out // NC

  def body(h_ref, a_ref, c_ref, wl_ref, wr_ref, b_ref, o_ref):
    inv = 1.0 / jnp.maximum(c_ref[:, 0:1], 1.0)
    acc = jnp.dot(a_ref[0], wl_ref[:dh], preferred_element_type=jnp.float32)
    acc += jnp.dot(a_ref[1], wl_ref[dh:], preferred_element_type=jnp.float32)
    res = acc * inv + b_ref[0]
    res += jnp.dot(h_ref[0], wr_ref[:dh], preferred_element_type=jnp.float32)
    res += jnp.dot(h_ref[1], wr_ref[dh:], preferred_element_type=jnp.float32)
    if relu:
      res = jnp.maximum(res, 0.0)
    o_ref[0] = res[:, :dho]
    o_ref[1] = res[:, dho:]

  return pl.pallas_call(
      body,
      grid=(n // bn,),
      in_specs=[
          pl.BlockSpec((NC, bn, dh), lambda i: (0, i, 0)),
          pl.BlockSpec((NC, bn, dh), lambda i: (0, i, 0)),
          pl.BlockSpec((bn, L), lambda i: (i, 0)),
          pl.BlockSpec(wl.shape, lambda i: (0, 0)),
          pl.BlockSpec(wr.shape, lambda i: (0, 0)),
          pl.BlockSpec((1, d_out), lambda i: (0, 0)),
      ],
      out_specs=pl.BlockSpec((NC, bn, dho), lambda i: (0, i, 0)),
      out_shape=jax.ShapeDtypeStruct((NC, n, dho), jnp.float32),
  )(h, agg, cnt, wl, wr, b)


def _tc_layer2(h, agg, cnt, wl, wr, b, w3l, w3r, b3, bn):
  _, n, dh = h.shape
  d3 = w3l.shape[1]
  dh3 = d3 // NC

  def body(h_ref, a_ref, c_ref, wl_ref, wr_ref, b_ref,
           w3l_ref, w3r_ref, b3_ref, t3_ref, r3_ref):
    inv = 1.0 / jnp.maximum(c_ref[:, 0:1], 1.0)
    acc = jnp.dot(a_ref[0], wl_ref[:dh], preferred_element_type=jnp.float32)
    acc += jnp.dot(a_ref[1], wl_ref[dh:], preferred_element_type=jnp.float32)
    res = acc * inv + b_ref[0]
    res += jnp.dot(h_ref[0], wr_ref[:dh], preferred_element_type=jnp.float32)
    res += jnp.dot(h_ref[1], wr_ref[dh:], preferred_element_type=jnp.float32)
    h2 = jnp.maximum(res, 0.0)
    t3 = jnp.dot(h2, w3l_ref[...], preferred_element_type=jnp.float32)
    t3_ref[0] = t3[:, :dh3]
    t3_ref[1] = t3[:, dh3:]
    r3_ref[...] = (
        jnp.dot(h2, w3r_ref[...], preferred_element_type=jnp.float32)
        + b3_ref[0])

  return pl.pallas_call(
      body,
      grid=(n // bn,),
      in_specs=[
          pl.BlockSpec((NC, bn, dh), lambda i: (0, i, 0)),
          pl.BlockSpec((NC, bn, dh), lambda i: (0, i, 0)),
          pl.BlockSpec((bn, L), lambda i: (i, 0)),
          pl.BlockSpec(wl.shape, lambda i: (0, 0)),
          pl.BlockSpec(wr.shape, lambda i: (0, 0)),
          pl.BlockSpec((1, wl.shape[1]), lambda i: (0, 0)),
          pl.BlockSpec(w3l.shape, lambda i: (0, 0)),
          pl.BlockSpec(w3r.shape, lambda i: (0, 0)),
          pl.BlockSpec((1, d3), lambda i: (0, 0)),
      ],
      out_specs=[
          pl.BlockSpec((NC, bn, dh3), lambda i: (0, i, 0)),
          pl.BlockSpec((bn, d3), lambda i: (i, 0)),
      ],
      out_shape=[
          jax.ShapeDtypeStruct((NC, n, dh3), jnp.float32),
          jax.ShapeDtypeStruct((n, d3), jnp.float32),
      ],
  )(h, agg, cnt, wl, wr, b, w3l, w3r, b3)


def _tc_layer3(agg, cnt, r3, bn):
  _, n, dh = agg.shape
  d = NC * dh

  def body(a_ref, c_ref, r_ref, o_ref):
    inv = 1.0 / jnp.maximum(c_ref[:, 0:1], 1.0)
    o = jnp.concatenate([a_ref[0], a_ref[1]], axis=1) * inv + r_ref[...]
    m = jnp.max(o, axis=-1, keepdims=True)
    e = o - m
    lse = jnp.log(jnp.sum(jnp.exp(e), axis=-1, keepdims=True))
    o_ref[...] = e - lse

  return pl.pallas_call(
      body,
      grid=(n // bn,),
      in_specs=[
          pl.BlockSpec((NC, bn, dh), lambda i: (0, i, 0)),
          pl.BlockSpec((bn, L), lambda i: (i, 0)),
          pl.BlockSpec((bn, d), lambda i: (i, 0)),
      ],
      out_specs=pl.BlockSpec((bn, d), lambda i: (i, 0)),
      out_shape=jax.ShapeDtypeStruct((n, d), jnp.float32),
  )(agg, cnt, r3)


def kernel(x, W1_l, b1, W1_r, W2_l, b2, W2_r, W3_l, b3, W3_r, edge_index):
  n, d_in = x.shape
  e = edge_index.shape[1]
  d_hid = W1_l.shape[1]
  d_out = W3_l.shape[1]

  n_chunks = -(-e // (NS * K * IB)) * IB
  e_pad = NS * K * n_chunks
  n_pad = -(-(n + 1) // (NS * WO)) * (NS * WO)
  bn = 1000 if n % 1000 == 0 else 8

  src = edge_index[0].astype(jnp.int32)
  dst = edge_index[1].astype(jnp.int32)
  src = jnp.concatenate([src, jnp.zeros((e_pad - e,), jnp.int32)])
  dst = jnp.concatenate([dst, jnp.full((e_pad - e,), n, jnp.int32)])
  src = src.reshape(NS, n_chunks, K)
  dst = dst.reshape(NS, n_chunks, K)
  srcs = jnp.stack([src + c * n for c in range(NC)])

  def split(a):
    return a.reshape(a.shape[0], NC, a.shape[1] // NC).transpose(1, 0, 2)

  zrow1 = jnp.zeros((WO, d_in // NC), jnp.float32)
  zrow2 = jnp.zeros((WO, d_hid // NC), jnp.float32)
  zrow3 = jnp.zeros((WO, d_out // NC), jnp.float32)
  zc = jnp.zeros((WO, L), jnp.float32)
  oc = jnp.ones((K, L), jnp.float32)

  xs = split(x)
  agg1, cnt = _make_segsum(n, n_pad, n_chunks, d_in // NC, True)(
      xs.reshape(NC * n, d_in // NC), srcs, dst, zrow1, zc, oc)
  cnt = cnt[:n]
  h1 = _tc_layer(xs, agg1[:, :n], cnt, W1_l, W1_r, b1.reshape(1, -1), bn,
                 relu=True)
  agg2 = _make_segsum(n, n_pad, n_chunks, d_hid // NC, False)(
      h1.reshape(NC * n, d_hid // NC), srcs, dst, zrow2)[0]
  t3, r3 = _tc_layer2(h1, agg2[:, :n], cnt, W2_l, W2_r, b2.reshape(1, -1),
                      W3_l, W3_r, b3.reshape(1, -1), bn)
  agg3 = _make_segsum(n, n_pad, n_chunks, d_out // NC, False)(
      t3.reshape(NC * n, d_out // NC), srcs, dst, zrow3)[0]
  return _tc_layer3(agg3[:, :n], cnt, r3, bn)


# speedup vs baseline: 4.7731x; 1.1413x over previous
import jax
import jax.numpy as jnp
from jax import lax
from jax.experimental import pallas as pl
from jax.experimental.pallas import tpu as pltpu
from jax.experimental.pallas import tpu_sc as plsc

NC = 2
NS = 16
L = 16
K = 128
IB = 16
WO = 64


def _make_segsum(n_nodes, n_pad, n_chunks, dh, with_cnt):
  rpt = n_pad // NS
  n_rounds = rpt // WO
  n_blocks = n_chunks // IB

  mesh = plsc.VectorSubcoreMesh(
      core_axis_name="c", subcore_axis_name="s",
      num_cores=NC, num_subcores=NS)

  def body(tbl, srcs, dsts, *rest):
    if with_cnt:
      (zrow, zc, oc, agg_h, cnt_h, src_v, dst_v, rows_a, rows_b, out_v,
       ones_v, cnt_ov, agg_sh, cnt_sh, sem_a, sem_b) = rest
    else:
      (zrow, agg_h, src_v, dst_v, rows_a, rows_b, out_v, agg_sh,
       sem_a, sem_b) = rest
    bufs = (rows_a, rows_b)
    sems = (sem_a, sem_b)
    cid = lax.axis_index("c")
    sid = lax.axis_index("s")
    base = sid * rpt

    pltpu.sync_copy(zrow, out_v)
    for r in range(n_rounds):
      pltpu.sync_copy(out_v, agg_sh.at[pl.ds(base + r * WO, WO)])
    if with_cnt:
      pltpu.sync_copy(zc, cnt_ov)
      for r in range(n_rounds):
        pltpu.sync_copy(cnt_ov, cnt_sh.at[pl.ds(base + r * WO, WO)])
      pltpu.sync_copy(oc, ones_v)
    plsc.subcore_barrier()

    def block(b, carry):
      pltpu.sync_copy(srcs.at[cid, sid, pl.ds(b * IB, IB)], src_v)
      pltpu.sync_copy(dsts.at[sid, pl.ds(b * IB, IB)], dst_v)
      cps = [None] * IB
      cps[0] = pltpu.async_copy(tbl.at[src_v.at[0]], bufs[0], sems[0])
      for i in range(IB):
        cps[i].wait()
        if i + 1 < IB:
          cps[i + 1] = pltpu.async_copy(
              tbl.at[src_v.at[i + 1]], bufs[(i + 1) % 2], sems[(i + 1) % 2])
        pltpu.sync_copy(bufs[i % 2], agg_sh.at[dst_v.at[i]], add=True)
        if with_cnt:
          pltpu.sync_copy(ones_v, cnt_sh.at[dst_v.at[i]], add=True)
      return carry
    lax.fori_loop(0, n_blocks, block, 0)
    plsc.subcore_barrier()

    for r in range(n_rounds):
      pltpu.sync_copy(agg_sh.at[pl.ds(base + r * WO, WO)], out_v)
      pltpu.sync_copy(out_v, agg_h.at[cid, pl.ds(base + r * WO, WO)])
    if with_cnt:
      @pl.when(cid == 0)
      def _():
        for r in range(n_rounds):
          pltpu.sync_copy(cnt_sh.at[pl.ds(base + r * WO, WO)], cnt_ov)
          pltpu.sync_copy(cnt_ov, cnt_h.at[pl.ds(base + r * WO, WO)])

  out_type = [jax.ShapeDtypeStruct((NC, n_pad, dh), jnp.float32)]
  scratch = [
      pltpu.VMEM((IB, K), jnp.int32),
      pltpu.VMEM((IB, K), jnp.int32),
      pltpu.VMEM((K, dh), jnp.float32),
      pltpu.VMEM((K, dh), jnp.float32),
      pltpu.VMEM((WO, dh), jnp.float32),
  ]
  if with_cnt:
    out_type.append(jax.ShapeDtypeStruct((n_pad, L), jnp.float32))
    scratch += [
        pltpu.VMEM((K, L), jnp.float32),
        pltpu.VMEM((WO, L), jnp.float32),
    ]
  scratch += [pltpu.VMEM_SHARED((n_pad, dh), jnp.float32)]
  if with_cnt:
    scratch += [pltpu.VMEM_SHARED((n_pad, L), jnp.float32)]
  scratch += [pltpu.SemaphoreType.DMA, pltpu.SemaphoreType.DMA]

  return pl.kernel(
      body, out_type=tuple(out_type), mesh=mesh,
      scratch_types=tuple(scratch),
      compiler_params=pltpu.CompilerParams(use_tc_tiling_on_sc=False))


def _tc_layer(h, agg, cnt, wl, wr, b, bn, relu):
  _, n, dh = h.shape
  d_out = wl.shape[1]
  dho = d_out // NC

  def body(h_ref, a_ref, c_ref, wl_ref, wr_ref, b_ref, o_ref):
    inv = 1.0 / jnp.maximum(c_ref[:, 0:1], 1.0)
    acc = jnp.dot(a_ref[0], wl_ref[:dh], preferred_element_type=jnp.float32)
    acc += jnp.dot(a_ref[1], wl_ref[dh:], preferred_element_type=jnp.float32)
    res = acc * inv + b_ref[0]
    res += jnp.dot(h_ref[0], wr_ref[:dh], preferred_element_type=jnp.float32)
    res += jnp.dot(h_ref[1], wr_ref[dh:], preferred_element_type=jnp.float32)
    if relu:
      res = jnp.maximum(res, 0.0)
    o_ref[0] = res[:, :dho]
    o_ref[1] = res[:, dho:]

  return pl.pallas_call(
      body,
      grid=(n // bn,),
      in_specs=[
          pl.BlockSpec((NC, bn, dh), lambda i: (0, i, 0)),
          pl.BlockSpec((NC, bn, dh), lambda i: (0, i, 0)),
          pl.BlockSpec((bn, L), lambda i: (i, 0)),
          pl.BlockSpec(wl.shape, lambda i: (0, 0)),
          pl.BlockSpec(wr.shape, lambda i: (0, 0)),
          pl.BlockSpec((1, d_out), lambda i: (0, 0)),
      ],
      out_specs=pl.BlockSpec((NC, bn, dho), lambda i: (0, i, 0)),
      out_shape=jax.ShapeDtypeStruct((NC, n, dho), jnp.float32),
  )(h, agg, cnt, wl, wr, b)


def _tc_layer2(h, agg, cnt, wl, wr, b, w3l, w3r, b3, bn):
  _, n, dh = h.shape
  d3 = w3l.shape[1]
  dh3 = d3 // NC

  def body(h_ref, a_ref, c_ref, wl_ref, wr_ref, b_ref,
           w3l_ref, w3r_ref, b3_ref, t3_ref, r3_ref):
    inv = 1.0 / jnp.maximum(c_ref[:, 0:1], 1.0)
    acc = jnp.dot(a_ref[0], wl_ref[:dh], preferred_element_type=jnp.float32)
    acc += jnp.dot(a_ref[1], wl_ref[dh:], preferred_element_type=jnp.float32)
    res = acc * inv + b_ref[0]
    res += jnp.dot(h_ref[0], wr_ref[:dh], preferred_element_type=jnp.float32)
    res += jnp.dot(h_ref[1], wr_ref[dh:], preferred_element_type=jnp.float32)
    h2 = jnp.maximum(res, 0.0)
    t3 = jnp.dot(h2, w3l_ref[...], preferred_element_type=jnp.float32)
    t3_ref[0] = t3[:, :dh3]
    t3_ref[1] = t3[:, dh3:]
    r3_ref[...] = (
        jnp.dot(h2, w3r_ref[...], preferred_element_type=jnp.float32)
        + b3_ref[0])

  return pl.pallas_call(
      body,
      grid=(n // bn,),
      in_specs=[
          pl.BlockSpec((NC, bn, dh), lambda i: (0, i, 0)),
          pl.BlockSpec((NC, bn, dh), lambda i: (0, i, 0)),
          pl.BlockSpec((bn, L), lambda i: (i, 0)),
          pl.BlockSpec(wl.shape, lambda i: (0, 0)),
          pl.BlockSpec(wr.shape, lambda i: (0, 0)),
          pl.BlockSpec((1, wl.shape[1]), lambda i: (0, 0)),
          pl.BlockSpec(w3l.shape, lambda i: (0, 0)),
          pl.BlockSpec(w3r.shape, lambda i: (0, 0)),
          pl.BlockSpec((1, d3), lambda i: (0, 0)),
      ],
      out_specs=[
          pl.BlockSpec((NC, bn, dh3), lambda i: (0, i, 0)),
          pl.BlockSpec((bn, d3), lambda i: (i, 0)),
      ],
      out_shape=[
          jax.ShapeDtypeStruct((NC, n, dh3), jnp.float32),
          jax.ShapeDtypeStruct((n, d3), jnp.float32),
      ],
  )(h, agg, cnt, wl, wr, b, w3l, w3r, b3)


def _tc_layer3(agg, cnt, r3, bn):
  _, n, dh = agg.shape
  d = NC * dh

  def body(a_ref, c_ref, r_ref, o_ref):
    inv = 1.0 / jnp.maximum(c_ref[:, 0:1], 1.0)
    o = jnp.concatenate([a_ref[0], a_ref[1]], axis=1) * inv + r_ref[...]
    m = jnp.max(o, axis=-1, keepdims=True)
    e = o - m
    lse = jnp.log(jnp.sum(jnp.exp(e), axis=-1, keepdims=True))
    o_ref[...] = e - lse

  return pl.pallas_call(
      body,
      grid=(n // bn,),
      in_specs=[
          pl.BlockSpec((NC, bn, dh), lambda i: (0, i, 0)),
          pl.BlockSpec((bn, L), lambda i: (i, 0)),
          pl.BlockSpec((bn, d), lambda i: (i, 0)),
      ],
      out_specs=pl.BlockSpec((bn, d), lambda i: (i, 0)),
      out_shape=jax.ShapeDtypeStruct((n, d), jnp.float32),
  )(agg, cnt, r3)


def kernel(x, W1_l, b1, W1_r, W2_l, b2, W2_r, W3_l, b3, W3_r, edge_index):
  n, d_in = x.shape
  e = edge_index.shape[1]
  d_hid = W1_l.shape[1]
  d_out = W3_l.shape[1]

  n_chunks = -(-e // (NS * K * IB)) * IB
  e_pad = NS * K * n_chunks
  n_pad = -(-(n + 1) // (NS * WO)) * (NS * WO)
  bn = 1000 if n % 1000 == 0 else 8

  src = edge_index[0].astype(jnp.int32)
  dst = edge_index[1].astype(jnp.int32)
  src = jnp.concatenate([src, jnp.zeros((e_pad - e,), jnp.int32)])
  dst = jnp.concatenate([dst, jnp.full((e_pad - e,), n, jnp.int32)])
  src = src.reshape(NS, n_chunks, K)
  dst = dst.reshape(NS, n_chunks, K)
  srcs = jnp.stack([src + c * n for c in range(NC)])

  def split(a):
    return a.reshape(a.shape[0], NC, a.shape[1] // NC).transpose(1, 0, 2)

  zrow1 = jnp.zeros((WO, d_in // NC), jnp.float32)
  zrow2 = jnp.zeros((WO, d_hid // NC), jnp.float32)
  zrow3 = jnp.zeros((WO, d_out // NC), jnp.float32)
  zc = jnp.zeros((WO, L), jnp.float32)
  oc = jnp.ones((K, L), jnp.float32)

  xs = split(x)
  agg1, cnt = _make_segsum(n, n_pad, n_chunks, d_in // NC, True)(
      xs.reshape(NC * n, d_in // NC), srcs, dst, zrow1, zc, oc)
  cnt = cnt[:n]
  h1 = _tc_layer(xs, agg1[:, :n], cnt, W1_l, W1_r, b1.reshape(1, -1), bn,
                 relu=True)
  agg2 = _make_segsum(n, n_pad, n_chunks, d_hid // NC, False)(
      h1.reshape(NC * n, d_hid // NC), srcs, dst, zrow2)[0]
  t3, r3 = _tc_layer2(h1, agg2[:, :n], cnt, W2_l, W2_r, b2.reshape(1, -1),
                      W3_l, W3_r, b3.reshape(1, -1), bn)
  agg3 = _make_segsum(n, n_pad, n_chunks, d_out // NC, False)(
      t3.reshape(NC * n, d_out // NC), srcs, dst, zrow3)[0]
  return _tc_layer3(agg3[:, :n], cnt, r3, bn)


# speedup vs baseline: 4.7821x; 1.0019x over previous
import jax
import jax.numpy as jnp
from jax import lax
from jax.experimental import pallas as pl
from jax.experimental.pallas import tpu as pltpu
from jax.experimental.pallas import tpu_sc as plsc

NC = 2
NS = 16
L = 16
K = 128
IB = 16
WO = 64


def _make_segsum(n_nodes, n_pad, n_chunks, dh, with_cnt):
  rpt = n_pad // NS
  n_rounds = rpt // WO
  n_blocks = n_chunks // IB

  mesh = plsc.VectorSubcoreMesh(
      core_axis_name="c", subcore_axis_name="s",
      num_cores=NC, num_subcores=NS)

  def body(tbl, srcs, dsts, *rest):
    if with_cnt:
      (zrow, zc, oc, agg_h, cnt_h, src_v, dst_v, rows_a, rows_b, out_v,
       ones_v, cnt_ov, agg_sh, cnt_sh, sem_a, sem_b, sem_sa, sem_sb) = rest
    else:
      (zrow, agg_h, src_v, dst_v, rows_a, rows_b, out_v, agg_sh,
       sem_a, sem_b, sem_sa, sem_sb) = rest
    bufs = (rows_a, rows_b)
    sems = (sem_a, sem_b)
    ssems = (sem_sa, sem_sb)
    cid = lax.axis_index("c")
    sid = lax.axis_index("s")
    base = sid * rpt

    pltpu.sync_copy(zrow, out_v)
    for r in range(n_rounds):
      pltpu.sync_copy(out_v, agg_sh.at[pl.ds(base + r * WO, WO)])
    if with_cnt:
      pltpu.sync_copy(zc, cnt_ov)
      for r in range(n_rounds):
        pltpu.sync_copy(cnt_ov, cnt_sh.at[pl.ds(base + r * WO, WO)])
      pltpu.sync_copy(oc, ones_v)
    plsc.subcore_barrier()

    def block(b, carry):
      pltpu.sync_copy(srcs.at[cid, sid, pl.ds(b * IB, IB)], src_v)
      pltpu.sync_copy(dsts.at[sid, pl.ds(b * IB, IB)], dst_v)
      gcp = [None] * IB
      scp = [None] * IB
      gcp[0] = pltpu.async_copy(tbl.at[src_v.at[0]], bufs[0], sems[0])
      for i in range(IB):
        gcp[i].wait()
        if i >= 1:
          scp[i - 1].wait()
        if i + 1 < IB:
          gcp[i + 1] = pltpu.async_copy(
              tbl.at[src_v.at[i + 1]], bufs[(i + 1) % 2], sems[(i + 1) % 2])
        scp[i] = pltpu.async_copy(
            bufs[i % 2], agg_sh.at[dst_v.at[i]], ssems[i % 2], add=True)
        if with_cnt:
          pltpu.sync_copy(ones_v, cnt_sh.at[dst_v.at[i]], add=True)
      scp[IB - 1].wait()
      return carry
    lax.fori_loop(0, n_blocks, block, 0)
    plsc.subcore_barrier()

    for r in range(n_rounds):
      pltpu.sync_copy(agg_sh.at[pl.ds(base + r * WO, WO)], out_v)
      pltpu.sync_copy(out_v, agg_h.at[cid, pl.ds(base + r * WO, WO)])
    if with_cnt:
      @pl.when(cid == 0)
      def _():
        for r in range(n_rounds):
          pltpu.sync_copy(cnt_sh.at[pl.ds(base + r * WO, WO)], cnt_ov)
          pltpu.sync_copy(cnt_ov, cnt_h.at[pl.ds(base + r * WO, WO)])

  out_type = [jax.ShapeDtypeStruct((NC, n_pad, dh), jnp.float32)]
  scratch = [
      pltpu.VMEM((IB, K), jnp.int32),
      pltpu.VMEM((IB, K), jnp.int32),
      pltpu.VMEM((K, dh), jnp.float32),
      pltpu.VMEM((K, dh), jnp.float32),
      pltpu.VMEM((WO, dh), jnp.float32),
  ]
  if with_cnt:
    out_type.append(jax.ShapeDtypeStruct((n_pad, L), jnp.float32))
    scratch += [
        pltpu.VMEM((K, L), jnp.float32),
        pltpu.VMEM((WO, L), jnp.float32),
    ]
  scratch += [pltpu.VMEM_SHARED((n_pad, dh), jnp.float32)]
  if with_cnt:
    scratch += [pltpu.VMEM_SHARED((n_pad, L), jnp.float32)]
  scratch += [pltpu.SemaphoreType.DMA] * 4

  return pl.kernel(
      body, out_type=tuple(out_type), mesh=mesh,
      scratch_types=tuple(scratch),
      compiler_params=pltpu.CompilerParams(use_tc_tiling_on_sc=False))


def _tc_layer(h, agg, cnt, wl, wr, b, bn, relu):
  _, n, dh = h.shape
  d_out = wl.shape[1]
  dho = d_out // NC

  def body(h_ref, a_ref, c_ref, wl_ref, wr_ref, b_ref, o_ref):
    inv = 1.0 / jnp.maximum(c_ref[:, 0:1], 1.0)
    acc = jnp.dot(a_ref[0], wl_ref[:dh], preferred_element_type=jnp.float32)
    acc += jnp.dot(a_ref[1], wl_ref[dh:], preferred_element_type=jnp.float32)
    res = acc * inv + b_ref[0]
    res += jnp.dot(h_ref[0], wr_ref[:dh], preferred_element_type=jnp.float32)
    res += jnp.dot(h_ref[1], wr_ref[dh:], preferred_element_type=jnp.float32)
    if relu:
      res = jnp.maximum(res, 0.0)
    o_ref[0] = res[:, :dho]
    o_ref[1] = res[:, dho:]

  return pl.pallas_call(
      body,
      grid=(n // bn,),
      in_specs=[
          pl.BlockSpec((NC, bn, dh), lambda i: (0, i, 0)),
          pl.BlockSpec((NC, bn, dh), lambda i: (0, i, 0)),
          pl.BlockSpec((bn, L), lambda i: (i, 0)),
          pl.BlockSpec(wl.shape, lambda i: (0, 0)),
          pl.BlockSpec(wr.shape, lambda i: (0, 0)),
          pl.BlockSpec((1, d_out), lambda i: (0, 0)),
      ],
      out_specs=pl.BlockSpec((NC, bn, dho), lambda i: (0, i, 0)),
      out_shape=jax.ShapeDtypeStruct((NC, n, dho), jnp.float32),
  )(h, agg, cnt, wl, wr, b)


def _tc_layer2(h, agg, cnt, wl, wr, b, w3l, w3r, b3, bn):
  _, n, dh = h.shape
  d3 = w3l.shape[1]
  dh3 = d3 // NC

  def body(h_ref, a_ref, c_ref, wl_ref, wr_ref, b_ref,
           w3l_ref, w3r_ref, b3_ref, t3_ref, r3_ref):
    inv = 1.0 / jnp.maximum(c_ref[:, 0:1], 1.0)
    acc = jnp.dot(a_ref[0], wl_ref[:dh], preferred_element_type=jnp.float32)
    acc += jnp.dot(a_ref[1], wl_ref[dh:], preferred_element_type=jnp.float32)
    res = acc * inv + b_ref[0]
    res += jnp.dot(h_ref[0], wr_ref[:dh], preferred_element_type=jnp.float32)
    res += jnp.dot(h_ref[1], wr_ref[dh:], preferred_element_type=jnp.float32)
    h2 = jnp.maximum(res, 0.0)
    t3 = jnp.dot(h2, w3l_ref[...], preferred_element_type=jnp.float32)
    t3_ref[0] = t3[:, :dh3]
    t3_ref[1] = t3[:, dh3:]
    r3_ref[...] = (
        jnp.dot(h2, w3r_ref[...], preferred_element_type=jnp.float32)
        + b3_ref[0])

  return pl.pallas_call(
      body,
      grid=(n // bn,),
      in_specs=[
          pl.BlockSpec((NC, bn, dh), lambda i: (0, i, 0)),
          pl.BlockSpec((NC, bn, dh), lambda i: (0, i, 0)),
          pl.BlockSpec((bn, L), lambda i: (i, 0)),
          pl.BlockSpec(wl.shape, lambda i: (0, 0)),
          pl.BlockSpec(wr.shape, lambda i: (0, 0)),
          pl.BlockSpec((1, wl.shape[1]), lambda i: (0, 0)),
          pl.BlockSpec(w3l.shape, lambda i: (0, 0)),
          pl.BlockSpec(w3r.shape, lambda i: (0, 0)),
          pl.BlockSpec((1, d3), lambda i: (0, 0)),
      ],
      out_specs=[
          pl.BlockSpec((NC, bn, dh3), lambda i: (0, i, 0)),
          pl.BlockSpec((bn, d3), lambda i: (i, 0)),
      ],
      out_shape=[
          jax.ShapeDtypeStruct((NC, n, dh3), jnp.float32),
          jax.ShapeDtypeStruct((n, d3), jnp.float32),
      ],
  )(h, agg, cnt, wl, wr, b, w3l, w3r, b3)


def _tc_layer3(agg, cnt, r3, bn):
  _, n, dh = agg.shape
  d = NC * dh

  def body(a_ref, c_ref, r_ref, o_ref):
    inv = 1.0 / jnp.maximum(c_ref[:, 0:1], 1.0)
    o = jnp.concatenate([a_ref[0], a_ref[1]], axis=1) * inv + r_ref[...]
    m = jnp.max(o, axis=-1, keepdims=True)
    e = o - m
    lse = jnp.log(jnp.sum(jnp.exp(e), axis=-1, keepdims=True))
    o_ref[...] = e - lse

  return pl.pallas_call(
      body,
      grid=(n // bn,),
      in_specs=[
          pl.BlockSpec((NC, bn, dh), lambda i: (0, i, 0)),
          pl.BlockSpec((bn, L), lambda i: (i, 0)),
          pl.BlockSpec((bn, d), lambda i: (i, 0)),
      ],
      out_specs=pl.BlockSpec((bn, d), lambda i: (i, 0)),
      out_shape=jax.ShapeDtypeStruct((n, d), jnp.float32),
  )(agg, cnt, r3)


def kernel(x, W1_l, b1, W1_r, W2_l, b2, W2_r, W3_l, b3, W3_r, edge_index):
  n, d_in = x.shape
  e = edge_index.shape[1]
  d_hid = W1_l.shape[1]
  d_out = W3_l.shape[1]

  n_chunks = -(-e // (NS * K * IB)) * IB
  e_pad = NS * K * n_chunks
  n_pad = -(-(n + 1) // (NS * WO)) * (NS * WO)
  bn = 1000 if n % 1000 == 0 else 8

  src = edge_index[0].astype(jnp.int32)
  dst = edge_index[1].astype(jnp.int32)
  src = jnp.concatenate([src, jnp.zeros((e_pad - e,), jnp.int32)])
  dst = jnp.concatenate([dst, jnp.full((e_pad - e,), n, jnp.int32)])
  src = src.reshape(NS, n_chunks, K)
  dst = dst.reshape(NS, n_chunks, K)
  srcs = jnp.stack([src + c * n for c in range(NC)])

  def split(a):
    return a.reshape(a.shape[0], NC, a.shape[1] // NC).transpose(1, 0, 2)

  zrow1 = jnp.zeros((WO, d_in // NC), jnp.float32)
  zrow2 = jnp.zeros((WO, d_hid // NC), jnp.float32)
  zrow3 = jnp.zeros((WO, d_out // NC), jnp.float32)
  zc = jnp.zeros((WO, L), jnp.float32)
  oc = jnp.ones((K, L), jnp.float32)

  xs = split(x)
  agg1, cnt = _make_segsum(n, n_pad, n_chunks, d_in // NC, True)(
      xs.reshape(NC * n, d_in // NC), srcs, dst, zrow1, zc, oc)
  cnt = cnt[:n]
  h1 = _tc_layer(xs, agg1[:, :n], cnt, W1_l, W1_r, b1.reshape(1, -1), bn,
                 relu=True)
  agg2 = _make_segsum(n, n_pad, n_chunks, d_hid // NC, False)(
      h1.reshape(NC * n, d_hid // NC), srcs, dst, zrow2)[0]
  t3, r3 = _tc_layer2(h1, agg2[:, :n], cnt, W2_l, W2_r, b2.reshape(1, -1),
                      W3_l, W3_r, b3.reshape(1, -1), bn)
  agg3 = _make_segsum(n, n_pad, n_chunks, d_out // NC, False)(
      t3.reshape(NC * n, d_out // NC), srcs, dst, zrow3)[0]
  return _tc_layer3(agg3[:, :n], cnt, r3, bn)


# speedup vs baseline: 6.8504x; 1.4325x over previous
import jax
import jax.numpy as jnp
from jax import lax
from jax.experimental import pallas as pl
from jax.experimental.pallas import tpu as pltpu
from jax.experimental.pallas import tpu_sc as plsc

NC = 2
NS = 16
L = 16
K = 128
IB = 16
WO = 64


BF = jnp.bfloat16


def _make_segsum(n_nodes, n_pad, n_chunks, dh, with_cnt):
  rpt = n_pad // NS
  n_rounds = rpt // WO
  n_blocks = n_chunks // IB

  mesh = plsc.VectorSubcoreMesh(
      core_axis_name="c", subcore_axis_name="s",
      num_cores=NC, num_subcores=NS)

  def body(tbl, srcs, dsts, *rest):
    if with_cnt:
      (zrow, zc, oc, agg_h, cnt_h, src_v, dst_v, rows_a, rows_b, out_v,
       ones_v, cnt_ov, agg_sh, cnt_sh, sem_a, sem_b, sem_sa, sem_sb) = rest
    else:
      (zrow, agg_h, src_v, dst_v, rows_a, rows_b, out_v, agg_sh,
       sem_a, sem_b, sem_sa, sem_sb) = rest
    bufs = (rows_a, rows_b)
    sems = (sem_a, sem_b)
    ssems = (sem_sa, sem_sb)
    cid = lax.axis_index("c")
    sid = lax.axis_index("s")
    base = sid * rpt

    pltpu.sync_copy(zrow, out_v)
    for r in range(n_rounds):
      pltpu.sync_copy(out_v, agg_sh.at[pl.ds(base + r * WO, WO)])
    if with_cnt:
      pltpu.sync_copy(zc, cnt_ov)
      for r in range(n_rounds):
        pltpu.sync_copy(cnt_ov, cnt_sh.at[pl.ds(base + r * WO, WO)])
      pltpu.sync_copy(oc, ones_v)
    plsc.subcore_barrier()

    def block(b, carry):
      pltpu.sync_copy(srcs.at[cid, sid, pl.ds(b * IB, IB)], src_v)
      pltpu.sync_copy(dsts.at[sid, pl.ds(b * IB, IB)], dst_v)
      gcp = [None] * IB
      scp = [None] * IB
      gcp[0] = pltpu.async_copy(tbl.at[src_v.at[0]], bufs[0], sems[0])
      for i in range(IB):
        gcp[i].wait()
        if i >= 1:
          scp[i - 1].wait()
        if i + 1 < IB:
          gcp[i + 1] = pltpu.async_copy(
              tbl.at[src_v.at[i + 1]], bufs[(i + 1) % 2], sems[(i + 1) % 2])
        scp[i] = pltpu.async_copy(
            bufs[i % 2], agg_sh.at[dst_v.at[i]], ssems[i % 2], add=True)
        if with_cnt:
          pltpu.sync_copy(ones_v, cnt_sh.at[dst_v.at[i]], add=True)
      scp[IB - 1].wait()
      return carry
    lax.fori_loop(0, n_blocks, block, 0)
    plsc.subcore_barrier()

    for r in range(n_rounds):
      pltpu.sync_copy(agg_sh.at[pl.ds(base + r * WO, WO)], out_v)
      pltpu.sync_copy(out_v, agg_h.at[cid, pl.ds(base + r * WO, WO)])
    if with_cnt:
      @pl.when(cid == 0)
      def _():
        for r in range(n_rounds):
          pltpu.sync_copy(cnt_sh.at[pl.ds(base + r * WO, WO)], cnt_ov)
          pltpu.sync_copy(cnt_ov, cnt_h.at[pl.ds(base + r * WO, WO)])

  out_type = [jax.ShapeDtypeStruct((NC, n_pad, dh), BF)]
  scratch = [
      pltpu.VMEM((IB, K), jnp.int32),
      pltpu.VMEM((IB, K), jnp.int32),
      pltpu.VMEM((K, dh), BF),
      pltpu.VMEM((K, dh), BF),
      pltpu.VMEM((WO, dh), BF),
  ]
  if with_cnt:
    out_type.append(jax.ShapeDtypeStruct((n_pad, L), jnp.float32))
    scratch += [
        pltpu.VMEM((K, L), jnp.float32),
        pltpu.VMEM((WO, L), jnp.float32),
    ]
  scratch += [pltpu.VMEM_SHARED((n_pad, dh), BF)]
  if with_cnt:
    scratch += [pltpu.VMEM_SHARED((n_pad, L), jnp.float32)]
  scratch += [pltpu.SemaphoreType.DMA] * 4

  return pl.kernel(
      body, out_type=tuple(out_type), mesh=mesh,
      scratch_types=tuple(scratch),
      compiler_params=pltpu.CompilerParams(use_tc_tiling_on_sc=False))


def _tc_layer(h, agg, cnt, wl, wr, b, bn, relu):
  _, n, dh = h.shape
  d_out = wl.shape[1]
  dho = d_out // NC

  def body(h_ref, a_ref, c_ref, wl_ref, wr_ref, b_ref, o_ref):
    inv = 1.0 / jnp.maximum(c_ref[:, 0:1], 1.0)
    a0 = a_ref[0].astype(jnp.float32)
    a1 = a_ref[1].astype(jnp.float32)
    acc = jnp.dot(a0, wl_ref[:dh], preferred_element_type=jnp.float32)
    acc += jnp.dot(a1, wl_ref[dh:], preferred_element_type=jnp.float32)
    res = acc * inv + b_ref[0]
    res += jnp.dot(h_ref[0], wr_ref[:dh], preferred_element_type=jnp.float32)
    res += jnp.dot(h_ref[1], wr_ref[dh:], preferred_element_type=jnp.float32)
    if relu:
      res = jnp.maximum(res, 0.0)
    o_ref[0] = res[:, :dho]
    o_ref[1] = res[:, dho:]

  return pl.pallas_call(
      body,
      grid=(n // bn,),
      in_specs=[
          pl.BlockSpec((NC, bn, dh), lambda i: (0, i, 0)),
          pl.BlockSpec((NC, bn, dh), lambda i: (0, i, 0)),
          pl.BlockSpec((bn, L), lambda i: (i, 0)),
          pl.BlockSpec(wl.shape, lambda i: (0, 0)),
          pl.BlockSpec(wr.shape, lambda i: (0, 0)),
          pl.BlockSpec((1, d_out), lambda i: (0, 0)),
      ],
      out_specs=pl.BlockSpec((NC, bn, dho), lambda i: (0, i, 0)),
      out_shape=jax.ShapeDtypeStruct((NC, n, dho), jnp.float32),
  )(h, agg, cnt, wl, wr, b)


def _tc_layer2(h, agg, cnt, wl, wr, b, w3l, w3r, b3, bn):
  _, n, dh = h.shape
  d3 = w3l.shape[1]
  dh3 = d3 // NC

  def body(h_ref, a_ref, c_ref, wl_ref, wr_ref, b_ref,
           w3l_ref, w3r_ref, b3_ref, t3_ref, r3_ref):
    inv = 1.0 / jnp.maximum(c_ref[:, 0:1], 1.0)
    a0 = a_ref[0].astype(jnp.float32)
    a1 = a_ref[1].astype(jnp.float32)
    acc = jnp.dot(a0, wl_ref[:dh], preferred_element_type=jnp.float32)
    acc += jnp.dot(a1, wl_ref[dh:], preferred_element_type=jnp.float32)
    res = acc * inv + b_ref[0]
    res += jnp.dot(h_ref[0], wr_ref[:dh], preferred_element_type=jnp.float32)
    res += jnp.dot(h_ref[1], wr_ref[dh:], preferred_element_type=jnp.float32)
    h2 = jnp.maximum(res, 0.0)
    t3 = jnp.dot(h2, w3l_ref[...], preferred_element_type=jnp.float32)
    t3_ref[0] = t3[:, :dh3]
    t3_ref[1] = t3[:, dh3:]
    r3_ref[...] = (
        jnp.dot(h2, w3r_ref[...], preferred_element_type=jnp.float32)
        + b3_ref[0])

  return pl.pallas_call(
      body,
      grid=(n // bn,),
      in_specs=[
          pl.BlockSpec((NC, bn, dh), lambda i: (0, i, 0)),
          pl.BlockSpec((NC, bn, dh), lambda i: (0, i, 0)),
          pl.BlockSpec((bn, L), lambda i: (i, 0)),
          pl.BlockSpec(wl.shape, lambda i: (0, 0)),
          pl.BlockSpec(wr.shape, lambda i: (0, 0)),
          pl.BlockSpec((1, wl.shape[1]), lambda i: (0, 0)),
          pl.BlockSpec(w3l.shape, lambda i: (0, 0)),
          pl.BlockSpec(w3r.shape, lambda i: (0, 0)),
          pl.BlockSpec((1, d3), lambda i: (0, 0)),
      ],
      out_specs=[
          pl.BlockSpec((NC, bn, dh3), lambda i: (0, i, 0)),
          pl.BlockSpec((bn, d3), lambda i: (i, 0)),
      ],
      out_shape=[
          jax.ShapeDtypeStruct((NC, n, dh3), jnp.float32),
          jax.ShapeDtypeStruct((n, d3), jnp.float32),
      ],
  )(h, agg, cnt, wl, wr, b, w3l, w3r, b3)


def _tc_layer3(agg, cnt, r3, bn):
  _, n, dh = agg.shape
  d = NC * dh

  def body(a_ref, c_ref, r_ref, o_ref):
    inv = 1.0 / jnp.maximum(c_ref[:, 0:1], 1.0)
    a = jnp.concatenate([a_ref[0], a_ref[1]], axis=1).astype(jnp.float32)
    o = a * inv + r_ref[...]
    m = jnp.max(o, axis=-1, keepdims=True)
    e = o - m
    lse = jnp.log(jnp.sum(jnp.exp(e), axis=-1, keepdims=True))
    o_ref[...] = e - lse

  return pl.pallas_call(
      body,
      grid=(n // bn,),
      in_specs=[
          pl.BlockSpec((NC, bn, dh), lambda i: (0, i, 0)),
          pl.BlockSpec((bn, L), lambda i: (i, 0)),
          pl.BlockSpec((bn, d), lambda i: (i, 0)),
      ],
      out_specs=pl.BlockSpec((bn, d), lambda i: (i, 0)),
      out_shape=jax.ShapeDtypeStruct((n, d), jnp.float32),
  )(agg, cnt, r3)


def kernel(x, W1_l, b1, W1_r, W2_l, b2, W2_r, W3_l, b3, W3_r, edge_index):
  n, d_in = x.shape
  e = edge_index.shape[1]
  d_hid = W1_l.shape[1]
  d_out = W3_l.shape[1]

  n_chunks = -(-e // (NS * K * IB)) * IB
  e_pad = NS * K * n_chunks
  n_pad = -(-(n + 1) // (NS * WO)) * (NS * WO)
  bn = 1000 if n % 1000 == 0 else 8

  src = edge_index[0].astype(jnp.int32)
  dst = edge_index[1].astype(jnp.int32)
  src = jnp.concatenate([src, jnp.zeros((e_pad - e,), jnp.int32)])
  dst = jnp.concatenate([dst, jnp.full((e_pad - e,), n, jnp.int32)])
  src = src.reshape(NS, n_chunks, K)
  dst = dst.reshape(NS, n_chunks, K)
  srcs = jnp.stack([src + c * n for c in range(NC)])

  def split(a):
    return a.reshape(a.shape[0], NC, a.shape[1] // NC).transpose(1, 0, 2)

  zrow1 = jnp.zeros((WO, d_in // NC), BF)
  zrow2 = jnp.zeros((WO, d_hid // NC), BF)
  zrow3 = jnp.zeros((WO, d_out // NC), BF)
  zc = jnp.zeros((WO, L), jnp.float32)
  oc = jnp.ones((K, L), jnp.float32)

  xs = split(x)
  agg1, cnt = _make_segsum(n, n_pad, n_chunks, d_in // NC, True)(
      xs.reshape(NC * n, d_in // NC).astype(BF), srcs, dst, zrow1, zc, oc)
  cnt = cnt[:n]
  h1 = _tc_layer(xs, agg1[:, :n], cnt, W1_l, W1_r, b1.reshape(1, -1), bn,
                 relu=True)
  agg2 = _make_segsum(n, n_pad, n_chunks, d_hid // NC, False)(
      h1.reshape(NC * n, d_hid // NC).astype(BF), srcs, dst, zrow2)[0]
  t3, r3 = _tc_layer2(h1, agg2[:, :n], cnt, W2_l, W2_r, b2.reshape(1, -1),
                      W3_l, W3_r, b3.reshape(1, -1), bn)
  agg3 = _make_segsum(n, n_pad, n_chunks, d_out // NC, False)(
      t3.reshape(NC * n, d_out // NC).astype(BF), srcs, dst, zrow3)[0]
  return _tc_layer3(agg3[:, :n], cnt, r3, bn)
